# Initial kernel scaffold; baseline (speedup 1.0000x reference)
import jax
import jax.numpy as jnp
from jax.experimental import pallas as pl


def kernel(beta, alpha, t):
    raise NotImplementedError("write your pallas kernel here")


# speedup vs baseline: 6.3246x; 6.3246x over previous
import functools

import jax
import jax.numpy as jnp
from jax import lax
from jax.experimental import pallas as pl
from jax.experimental.pallas import tpu as pltpu
from jax.experimental.pallas import tpu_sc as plsc

BATCH = 16384
NC = 2
NS = 16
NW = NC * NS
W = 128
ROWS = BATCH // W
RPW = ROWS // NW


def _sc_gather_body(beta_hbm, alpha_hbm, t_hbm, beta_out, alpha_out,
                    idx_v, outb_v, outa_v, semb, sema):
    wid = lax.axis_index("s") * NC + lax.axis_index("c")
    r0 = wid * RPW
    pltpu.sync_copy(t_hbm.at[pl.ds(r0, RPW)], idx_v)
    copies = []
    for j in range(RPW):
        copies.append(
            pltpu.async_copy(beta_hbm.at[idx_v.at[j]], outb_v.at[j], semb))
        copies.append(
            pltpu.async_copy(alpha_hbm.at[idx_v.at[j]], outa_v.at[j], sema))
    for c in copies:
        c.wait()
    pltpu.sync_copy(outb_v, beta_out.at[pl.ds(r0, RPW)])
    pltpu.sync_copy(outa_v, alpha_out.at[pl.ds(r0, RPW)])


@jax.jit
def _sc_gather(beta, alpha, t):
    mesh = plsc.VectorSubcoreMesh(core_axis_name="c", subcore_axis_name="s")
    f = functools.partial(
        pl.kernel,
        mesh=mesh,
        out_type=(
            jax.ShapeDtypeStruct((ROWS, W), jnp.float32),
            jax.ShapeDtypeStruct((ROWS, W), jnp.float32),
        ),
        scratch_types=[
            pltpu.VMEM((RPW, W), jnp.int32),
            pltpu.VMEM((RPW, W), jnp.float32),
            pltpu.VMEM((RPW, W), jnp.float32),
            pltpu.SemaphoreType.DMA,
            pltpu.SemaphoreType.DMA,
        ],
    )(_sc_gather_body)
    return f(beta, alpha, t.reshape(ROWS, W))


def kernel(beta, alpha, t):
    beta_t, alpha_t = _sc_gather(beta, alpha, t.astype(jnp.int32))
    return (beta_t.reshape(BATCH), alpha_t.reshape(BATCH))


# speedup vs baseline: 8.5804x; 1.3567x over previous
import functools

import jax
import jax.numpy as jnp
from jax import lax
from jax.experimental import pallas as pl
from jax.experimental.pallas import tpu as pltpu
from jax.experimental.pallas import tpu_sc as plsc

BATCH = 16384
NC = 2
NS = 16
NW = NC * NS
W = 128
ROWS = BATCH // W
RPW = ROWS // NW


def _sc_gather_body(beta_hbm, alpha_hbm, t_hbm, beta_out, alpha_out,
                    beta_v, alpha_v, idx_v, outb_v, outa_v, semb, sema):
    sid = lax.axis_index("s")
    wid = sid * NC + lax.axis_index("c")
    r0 = wid * RPW

    @pl.when(sid == 0)
    def _stage_tables():
        pltpu.sync_copy(beta_hbm, beta_v)
        pltpu.sync_copy(alpha_hbm, alpha_v)

    pltpu.sync_copy(t_hbm.at[pl.ds(r0, RPW)], idx_v)
    plsc.subcore_barrier()
    copies = []
    for j in range(RPW):
        copies.append(
            pltpu.async_copy(beta_v.at[idx_v.at[j]], outb_v.at[j], semb))
        copies.append(
            pltpu.async_copy(alpha_v.at[idx_v.at[j]], outa_v.at[j], sema))
    for c in copies:
        c.wait()
    pltpu.sync_copy(outb_v, beta_out.at[pl.ds(r0, RPW)])
    pltpu.sync_copy(outa_v, alpha_out.at[pl.ds(r0, RPW)])


@jax.jit
def _sc_gather(beta, alpha, t):
    mesh = plsc.VectorSubcoreMesh(core_axis_name="c", subcore_axis_name="s")
    f = functools.partial(
        pl.kernel,
        mesh=mesh,
        out_type=(
            jax.ShapeDtypeStruct((ROWS, W), jnp.float32),
            jax.ShapeDtypeStruct((ROWS, W), jnp.float32),
        ),
        scratch_types=[
            pltpu.VMEM_SHARED((1000,), jnp.float32),
            pltpu.VMEM_SHARED((1000,), jnp.float32),
            pltpu.VMEM((RPW, W), jnp.int32),
            pltpu.VMEM((RPW, W), jnp.float32),
            pltpu.VMEM((RPW, W), jnp.float32),
            pltpu.SemaphoreType.DMA,
            pltpu.SemaphoreType.DMA,
        ],
    )(_sc_gather_body)
    return f(beta, alpha, t.reshape(ROWS, W))


def kernel(beta, alpha, t):
    beta_t, alpha_t = _sc_gather(beta, alpha, t.astype(jnp.int32))
    return (beta_t.reshape(BATCH), alpha_t.reshape(BATCH))


# speedup vs baseline: 8.6521x; 1.0084x over previous
import functools

import jax
import jax.numpy as jnp
from jax import lax
from jax.experimental import pallas as pl
from jax.experimental.pallas import tpu as pltpu
from jax.experimental.pallas import tpu_sc as plsc

BATCH = 16384
NC = 2
NS = 16
NW = NC * NS
BPW = BATCH // NW


def _sc_gather_body(beta_hbm, alpha_hbm, t_hbm, beta_out, alpha_out,
                    beta_v, alpha_v, idx_v, outb_v, outa_v, semb, sema):
    sid = lax.axis_index("s")
    wid = sid * NC + lax.axis_index("c")
    b0 = wid * BPW

    @pl.when(sid == 0)
    def _stage_tables():
        pltpu.sync_copy(beta_hbm, beta_v)
        pltpu.sync_copy(alpha_hbm, alpha_v)

    pltpu.sync_copy(t_hbm.at[pl.ds(b0, BPW)], idx_v)
    plsc.subcore_barrier()
    cb = pltpu.async_copy(beta_v.at[idx_v], outb_v, semb)
    ca = pltpu.async_copy(alpha_v.at[idx_v], outa_v, sema)
    cb.wait()
    ca.wait()
    pltpu.sync_copy(outb_v, beta_out.at[pl.ds(b0, BPW)])
    pltpu.sync_copy(outa_v, alpha_out.at[pl.ds(b0, BPW)])


@jax.jit
def _sc_gather(beta, alpha, t):
    mesh = plsc.VectorSubcoreMesh(core_axis_name="c", subcore_axis_name="s")
    f = functools.partial(
        pl.kernel,
        mesh=mesh,
        out_type=(
            jax.ShapeDtypeStruct((BATCH,), jnp.float32),
            jax.ShapeDtypeStruct((BATCH,), jnp.float32),
        ),
        scratch_types=[
            pltpu.VMEM_SHARED((1000,), jnp.float32),
            pltpu.VMEM_SHARED((1000,), jnp.float32),
            pltpu.VMEM((BPW,), jnp.int32),
            pltpu.VMEM((BPW,), jnp.float32),
            pltpu.VMEM((BPW,), jnp.float32),
            pltpu.SemaphoreType.DMA,
            pltpu.SemaphoreType.DMA,
        ],
    )(_sc_gather_body)
    return f(beta, alpha, t)


def kernel(beta, alpha, t):
    beta_t, alpha_t = _sc_gather(beta, alpha, t.astype(jnp.int32))
    return (beta_t, alpha_t)


# speedup vs baseline: 9.1450x; 1.0570x over previous
import functools

import jax
import jax.numpy as jnp
from jax import lax
from jax.experimental import pallas as pl
from jax.experimental.pallas import tpu as pltpu
from jax.experimental.pallas import tpu_sc as plsc

NUM_T = 1000
BATCH = 16384
NC = 2
NS = 16
NW = NC * NS
BPW = BATCH // NW


def _sc_gather_body(beta_hbm, alpha_hbm, t_hbm, beta_out, alpha_out,
                    beta_v, alpha_v, idx_v, outb_v, outa_v,
                    semt, semb, sema, semo):
    sid = lax.axis_index("s")
    wid = sid * NC + lax.axis_index("c")
    b0 = wid * BPW

    ct = pltpu.async_copy(t_hbm.at[pl.ds(b0, BPW)], idx_v, semt)

    @pl.when(sid == 0)
    def _stage_tables():
        cb = pltpu.async_copy(beta_hbm, beta_v, semb)
        ca = pltpu.async_copy(alpha_hbm, alpha_v, sema)
        cb.wait()
        ca.wait()

    plsc.subcore_barrier()
    ct.wait()
    cb = pltpu.async_copy(beta_v.at[idx_v], outb_v, semb)
    ca = pltpu.async_copy(alpha_v.at[idx_v], outa_v, sema)
    cb.wait()
    ca.wait()
    co1 = pltpu.async_copy(outb_v, beta_out.at[pl.ds(b0, BPW)], semo)
    co2 = pltpu.async_copy(outa_v, alpha_out.at[pl.ds(b0, BPW)], semo)
    co1.wait()
    co2.wait()


@jax.jit
def _sc_gather(beta, alpha, t):
    mesh = plsc.VectorSubcoreMesh(core_axis_name="c", subcore_axis_name="s")
    f = functools.partial(
        pl.kernel,
        mesh=mesh,
        out_type=(
            jax.ShapeDtypeStruct((BATCH,), jnp.float32),
            jax.ShapeDtypeStruct((BATCH,), jnp.float32),
        ),
        scratch_types=[
            pltpu.VMEM_SHARED((NUM_T,), jnp.float32),
            pltpu.VMEM_SHARED((NUM_T,), jnp.float32),
            pltpu.VMEM((BPW,), jnp.int32),
            pltpu.VMEM((BPW,), jnp.float32),
            pltpu.VMEM((BPW,), jnp.float32),
            pltpu.SemaphoreType.DMA,
            pltpu.SemaphoreType.DMA,
            pltpu.SemaphoreType.DMA,
            pltpu.SemaphoreType.DMA,
        ],
    )(_sc_gather_body)
    return f(beta, alpha, t)


def kernel(beta, alpha, t):
    beta_t, alpha_t = _sc_gather(beta, alpha, t.astype(jnp.int32))
    return (beta_t, alpha_t)
